# Initial kernel scaffold; baseline (speedup 1.0000x reference)
import jax
import jax.numpy as jnp
from jax.experimental import pallas as pl


def kernel(features, positions, gene_ids, mask, original_sample_indices, gene_table, W_feat, w_cls):
    raise NotImplementedError("write your pallas kernel here")


# speedup vs baseline: 2.0192x; 2.0192x over previous
import functools

import jax
import jax.numpy as jnp
from jax import lax
from jax.experimental import pallas as pl
from jax.experimental.pallas import tpu as pltpu
from jax.experimental.pallas import tpu_sc as plsc

B = 2048
V = 512
FEAT_DIM = 16
D_MODEL = 64
NUM_SAMPLES = 256

N = B * V
GATHER_W = 128
NUM_IDX_ROWS = N // GATHER_W


def _make_sc_gather():
    info = plsc.get_sparse_core_info()
    nc, ns = info.num_cores, info.num_subcores
    nw = nc * ns
    rows_per_w = NUM_IDX_ROWS // nw

    mesh = plsc.VectorSubcoreMesh(core_axis_name="c", subcore_axis_name="s")

    @functools.partial(
        pl.kernel,
        mesh=mesh,
        compiler_params=pltpu.CompilerParams(use_tc_tiling_on_sc=False),
        out_type=jax.ShapeDtypeStruct((N, D_MODEL), jnp.float32),
        scratch_types=[
            pltpu.VMEM((GATHER_W,), jnp.int32),
            pltpu.VMEM((GATHER_W, D_MODEL), jnp.float32),
            pltpu.SemaphoreType.DMA,
        ],
    )
    def sc_gather(idx_hbm, table_hbm, out_hbm, idx_v, rows_v, sem):
        wid = lax.axis_index("s") * nc + lax.axis_index("c")
        base = wid * rows_per_w

        def body(j, carry):
            r = base + j
            pltpu.sync_copy(idx_hbm.at[r], idx_v)
            pltpu.async_copy(table_hbm.at[idx_v], rows_v, sem).wait()
            pltpu.sync_copy(rows_v, out_hbm.at[pl.ds(r * GATHER_W, GATHER_W)])
            return carry

        lax.fori_loop(0, rows_per_w, body, 0)

    return sc_gather


_NB = 8
_R = _NB * V


def _tanh(x):
    x = jnp.clip(x, -7.90531110763549805, 7.90531110763549805)
    x2 = x * x
    p = jnp.float32(-2.76076847742355e-16)
    p = p * x2 + jnp.float32(2.00018790482477e-13)
    p = p * x2 + jnp.float32(-8.60467152213735e-11)
    p = p * x2 + jnp.float32(5.12229709037114e-08)
    p = p * x2 + jnp.float32(1.48572235717979e-05)
    p = p * x2 + jnp.float32(6.37261928875436e-04)
    p = p * x2 + jnp.float32(4.89352455891786e-03)
    p = p * x
    q = jnp.float32(1.19825839466702e-06)
    q = q * x2 + jnp.float32(1.18534705686654e-04)
    q = q * x2 + jnp.float32(2.26843463243900e-03)
    q = q * x2 + jnp.float32(4.89352518554385e-03)
    return p / q


def _tc_body(ge_ref, ft_ref, ps_ref, mk_ref, sid_ref, w_ref, wc_ref,
             out_ref, accs, accc):
    i = pl.program_id(0)

    @pl.when(i == 0)
    def _init():
        accs[...] = jnp.zeros_like(accs)
        accc[...] = jnp.zeros_like(accc)

    fp = jnp.dot(ft_ref[...], w_ref[...], preferred_element_type=jnp.float32,
                 precision=jax.lax.Precision.HIGHEST)
    x = ge_ref[...] + fp + ps_ref[...] * 1e-5
    h = _tanh(x)
    t = h * wc_ref[...] * mk_ref[...]
    num = t.reshape(_NB, V, D_MODEL).sum(axis=1)
    num = num.sum(axis=1, keepdims=True)
    den = mk_ref[...].reshape(_NB, V, 1).sum(axis=1)
    logit = num / jnp.maximum(den, 1.0)

    ids = sid_ref[...]
    lanes = lax.broadcasted_iota(jnp.int32, (_NB, NUM_SAMPLES), 1)
    oh = (ids == lanes).astype(jnp.float32)
    accs[...] += oh * logit
    accc[...] += oh

    @pl.when(i == pl.num_programs(0) - 1)
    def _fin():
        ssum = accs[...].sum(axis=0, keepdims=True)
        csum = accc[...].sum(axis=0, keepdims=True)
        out_ref[...] = ssum / jnp.maximum(csum, 1.0)


def _tc_call(ge, ft, ps, mk, sid, W, wc):
    grid = B // _NB
    return pl.pallas_call(
        _tc_body,
        grid=(grid,),
        in_specs=[
            pl.BlockSpec((_R, D_MODEL), lambda i: (i, 0)),
            pl.BlockSpec((_R, FEAT_DIM), lambda i: (i, 0)),
            pl.BlockSpec((_R, 1), lambda i: (i, 0)),
            pl.BlockSpec((_R, 1), lambda i: (i, 0)),
            pl.BlockSpec((_NB, 1), lambda i: (i, 0)),
            pl.BlockSpec((FEAT_DIM, D_MODEL), lambda i: (0, 0)),
            pl.BlockSpec((1, D_MODEL), lambda i: (0, 0)),
        ],
        out_specs=pl.BlockSpec((1, NUM_SAMPLES), lambda i: (0, 0)),
        out_shape=jax.ShapeDtypeStruct((1, NUM_SAMPLES), jnp.float32),
        scratch_shapes=[
            pltpu.VMEM((_NB, NUM_SAMPLES), jnp.float32),
            pltpu.VMEM((_NB, NUM_SAMPLES), jnp.float32),
        ],
    )(ge, ft, ps, mk, sid, W, wc)


def kernel(features, positions, gene_ids, mask, original_sample_indices,
           gene_table, W_feat, w_cls):
    ids2d = gene_ids.reshape(NUM_IDX_ROWS, GATHER_W).astype(jnp.int32)
    ge = _make_sc_gather()(ids2d, gene_table)

    ft = features.reshape(N, FEAT_DIM)
    ps = positions.reshape(N, 1).astype(jnp.float32)
    mk = mask.reshape(N, 1).astype(jnp.float32)
    sid = original_sample_indices.reshape(B, 1).astype(jnp.int32)
    wc = w_cls.reshape(1, D_MODEL)

    out = _tc_call(ge, ft, ps, mk, sid, W_feat, wc)
    return out.reshape(NUM_SAMPLES)
